# baseline (speedup 1.0000x reference)
import functools

import jax
import jax.numpy as jnp
from jax import lax
from jax.experimental import pallas as pl
from jax.experimental.pallas import tpu as pltpu
from jax.experimental.pallas import tpu_sc as plsc

_HIDDEN = 64
_SCALE = 8.0
_VOCAB = 1000000
_SPLIT = 524288
_NC = 2
_NS = 16
_NW = _NC * _NS
_BT = 4096 // 128
_J = 200
_NB = 4
_JG = _J // _NB

_P1W = 1024
_P1G = _SPLIT // _P1W
_P1LAST = (_VOCAB - 1) // _P1W


def _pack_body(a_ref, b_ref, o_ref):
    o_ref[...] = jnp.concatenate([a_ref[...], b_ref[...]], axis=0).T


_pack_table = pl.pallas_call(
    _pack_body,
    grid=(_P1G,),
    in_specs=[
        pl.BlockSpec((_HIDDEN, _P1W), lambda g: (0, g)),
        pl.BlockSpec(
            (_HIDDEN, _P1W),
            lambda g: (0, jnp.minimum(_P1G + g, _P1LAST)),
        ),
    ],
    out_specs=pl.BlockSpec((_P1W, 128), lambda g: (g, 0)),
    out_shape=jax.ShapeDtypeStruct((_SPLIT, 128), jnp.float32),
)

_mesh = plsc.VectorSubcoreMesh(core_axis_name="c", subcore_axis_name="s")


@functools.partial(
    pl.kernel,
    out_type=jax.ShapeDtypeStruct((_J, _BT, 64, 128), jnp.float32),
    mesh=_mesh,
    scratch_types=(
        [pltpu.VMEM((_J, 128), jnp.int32)]
        + [pltpu.VMEM((128,), jnp.int32)] * _NB
        + [pltpu.VMEM((128, 128), jnp.float32)] * _NB
        + [pltpu.VMEM((64, 128), jnp.float32)] * _NB
        + [pltpu.SemaphoreType.DMA] * (2 * _NB)
    ),
)
def _gather_halves(idx_hbm, table_hbm, out_hbm, idx_v, *rest):
    idx_g = rest[:_NB]
    buf_g = rest[_NB:2 * _NB]
    buf_c = rest[2 * _NB:3 * _NB]
    sem_g = rest[3 * _NB:4 * _NB]
    sem_s = rest[4 * _NB:5 * _NB]

    wid = lax.axis_index("s") * _NC + lax.axis_index("c")

    pltpu.sync_copy(idx_hbm.at[:, wid], idx_v)

    def stage_and_issue_gather(j, b):
        for g in range(8):
            sl = pl.ds(16 * g, 16)
            idx_g[b][sl] = idx_v[j, sl] & (_SPLIT - 1)
        pltpu.async_copy(table_hbm.at[idx_g[b]], buf_g[b], sem_g[b])

    def wait_gather(b):
        pltpu.make_async_copy(
            table_hbm.at[pl.ds(0, 128)], buf_g[b], sem_g[b]
        ).wait()

    def wait_scatter(b):
        pltpu.make_async_copy(buf_c[b], out_hbm.at[0, 0], sem_s[b]).wait()

    def compact_scale(j, b):
        for g in range(8):
            pvec = lax.shift_right_logical(idx_v[j, pl.ds(16 * g, 16)], 19)
            pvec = pvec * 64
            for l0 in range(16):
                l = 16 * g + l0
                c0 = pvec[l0]
                row = l % 64
                half = (l // 64) * 64
                for q in range(4):
                    v = buf_g[b][l, pl.ds(c0 + 16 * q, 16)] * _SCALE
                    buf_c[b][row, pl.ds(half + 16 * q, 16)] = v

    for b in range(_NB):
        stage_and_issue_gather(b, b)

    @pl.loop(0, _JG)
    def _group(t):
        for k in range(_NB):
            j = t * _NB + k
            wait_gather(k)

            @pl.when(t > 0)
            def _():
                wait_scatter(k)

            compact_scale(j, k)

            @pl.when(t + 1 < _JG)
            def _():
                stage_and_issue_gather(j + _NB, k)

            pltpu.async_copy(buf_c[k], out_hbm.at[j, wid], sem_s[k])

    for b in range(_NB):
        wait_scatter(b)


_JB = 25


def _transpose_body(i_ref, o_ref):
    blk = i_ref[:, 0].reshape(_JB * 64, 128)
    t = blk.T
    outs = []
    for jj in range(_JB):
        sl = slice(64 * jj, 64 * jj + 64)
        outs.append(jnp.concatenate([t[:64, sl], t[64:, sl]], axis=1))
    out = jnp.stack(outs, axis=0)
    o_ref[...] = out.reshape(_JB, 8, 1, 8, 128)


_to_final = pl.pallas_call(
    _transpose_body,
    grid=(_J // _JB, _BT),
    in_specs=[pl.BlockSpec((_JB, 1, 64, 128), lambda j, b: (j, b, 0, 0))],
    out_specs=pl.BlockSpec(
        (_JB, 8, 1, 8, 128), lambda j, b: (j, 0, b, 0, 0)
    ),
    out_shape=jax.ShapeDtypeStruct((_J, 8, _BT, 8, 128), jnp.float32),
)


def kernel(x, table):
    table2 = _pack_table(table.T, table.T)
    idx = x.T.reshape(_J, _BT, 128)
    out_sc = _gather_halves(idx, table2)
    out5 = _to_final(out_sc)
    return out5.transpose(2, 4, 0, 1, 3).reshape(4096, _J, _HIDDEN)


# speedup vs baseline: 1.6302x; 1.6302x over previous
import functools

import jax
import jax.numpy as jnp
from jax import lax
from jax.experimental import pallas as pl
from jax.experimental.pallas import tpu as pltpu
from jax.experimental.pallas import tpu_sc as plsc

_HIDDEN = 64
_SCALE = 8.0
_VOCAB = 1000000
_SPLIT = 524288
_NC = 2
_NS = 16
_NW = _NC * _NS
_BT = 4096 // 128
_J = 200
_NB = 4
_JG = _J // _NB

_P1W = 8192
_P1G = _SPLIT // _P1W
_P1LAST = (_VOCAB - 1) // _P1W


def _pack_body(a_ref, b_ref, o_ref):
    o_ref[...] = jnp.concatenate([a_ref[...], b_ref[...]], axis=0).T


_pack_table = pl.pallas_call(
    _pack_body,
    grid=(_P1G,),
    in_specs=[
        pl.BlockSpec((_HIDDEN, _P1W), lambda g: (0, g)),
        pl.BlockSpec(
            (_HIDDEN, _P1W),
            lambda g: (0, jnp.minimum(_P1G + g, _P1LAST)),
        ),
    ],
    out_specs=pl.BlockSpec((_P1W, 128), lambda g: (g, 0)),
    out_shape=jax.ShapeDtypeStruct((_SPLIT, 128), jnp.float32),
)

_mesh = plsc.VectorSubcoreMesh(core_axis_name="c", subcore_axis_name="s")


@functools.partial(
    pl.kernel,
    out_type=jax.ShapeDtypeStruct((_J, _BT, 128, 128), jnp.float32),
    mesh=_mesh,
    scratch_types=(
        [pltpu.VMEM((_J, 128), jnp.int32)]
        + [pltpu.VMEM((128,), jnp.int32)] * _NB
        + [pltpu.VMEM((128, 128), jnp.float32)] * _NB
        + [pltpu.SemaphoreType.DMA] * (2 * _NB)
    ),
)
def _gather_rows(idx_hbm, table_hbm, out_hbm, idx_v, *rest):
    idx_g = rest[:_NB]
    buf_g = rest[_NB:2 * _NB]
    sem_g = rest[2 * _NB:3 * _NB]
    sem_s = rest[3 * _NB:4 * _NB]

    wid = lax.axis_index("s") * _NC + lax.axis_index("c")

    pltpu.sync_copy(idx_hbm.at[:, wid], idx_v)

    def stage_and_issue_gather(j, b):
        for g in range(8):
            sl = pl.ds(16 * g, 16)
            idx_g[b][sl] = idx_v[j, sl] & (_SPLIT - 1)
        pltpu.async_copy(table_hbm.at[idx_g[b]], buf_g[b], sem_g[b])

    def wait_gather(b):
        pltpu.make_async_copy(
            table_hbm.at[pl.ds(0, 128)], buf_g[b], sem_g[b]
        ).wait()

    def wait_scatter(b):
        pltpu.make_async_copy(buf_g[b], out_hbm.at[0, 0], sem_s[b]).wait()

    for b in range(_NB):
        stage_and_issue_gather(b, b)

    @pl.loop(0, _JG)
    def _group(t):
        for k in range(_NB):
            j = t * _NB + k
            wait_gather(k)
            pltpu.async_copy(buf_g[k], out_hbm.at[j, wid], sem_s[k])

            @pl.when(t + 1 < _JG)
            def _():
                wait_scatter(k)
                stage_and_issue_gather(j + _NB, k)

    for k in range(_NB):
        wait_scatter(k)


_JB = 25


def _transpose_body(i_ref, idx_ref, o_ref):
    blk = i_ref[:, 0].reshape(_JB * 128, 128)
    t = blk.T
    pb = pl.program_id(1)
    outs = []
    for jj in range(_JB):
        sub = t[:, 128 * jj:128 * jj + 128]
        row = idx_ref[jj, pb, :]
        m = (row >= _SPLIT)[None, :]
        sel = jnp.where(m, sub[64:, :], sub[:64, :])
        outs.append(sel * _SCALE)
    out = jnp.stack(outs, axis=0)
    o_ref[...] = out.reshape(_JB, 8, 1, 8, 128)


_to_final = pl.pallas_call(
    _transpose_body,
    grid=(_J // _JB, _BT),
    in_specs=[
        pl.BlockSpec((_JB, 1, 128, 128), lambda j, b: (j, b, 0, 0)),
        pl.BlockSpec((_JB, _BT, 128), lambda j, b: (j, 0, 0)),
    ],
    out_specs=pl.BlockSpec(
        (_JB, 8, 1, 8, 128), lambda j, b: (j, 0, b, 0, 0)
    ),
    out_shape=jax.ShapeDtypeStruct((_J, 8, _BT, 8, 128), jnp.float32),
)


def kernel(x, table):
    table2 = _pack_table(table.T, table.T)
    idx = x.T.reshape(_J, _BT, 128)
    out_sc = _gather_rows(idx, table2)
    out5 = _to_final(out_sc, idx)
    return out5.transpose(2, 4, 0, 1, 3).reshape(4096, _J, _HIDDEN)


# speedup vs baseline: 1.8213x; 1.1172x over previous
import functools

import jax
import jax.numpy as jnp
from jax import lax
from jax.experimental import pallas as pl
from jax.experimental.pallas import tpu as pltpu
from jax.experimental.pallas import tpu_sc as plsc

_HIDDEN = 64
_SCALE = 8.0
_VOCAB = 1000000
_SPLIT = 524288
_NC = 2
_NS = 16
_NW = _NC * _NS
_BT = 4096 // 128
_J = 200
_NB = 4
_JG = _J // _NB

_P1W = 8192
_P1G = _SPLIT // _P1W
_P1LAST = (_VOCAB - 1) // _P1W


def _pack_body(a_ref, b_ref, o_ref):
    o_ref[...] = jnp.concatenate([a_ref[...], b_ref[...]], axis=0).T


_pack_table = pl.pallas_call(
    _pack_body,
    grid=(_P1G,),
    in_specs=[
        pl.BlockSpec((_HIDDEN, _P1W), lambda g: (0, g)),
        pl.BlockSpec(
            (_HIDDEN, _P1W),
            lambda g: (0, jnp.minimum(_P1G + g, _P1LAST)),
        ),
    ],
    out_specs=pl.BlockSpec((_P1W, 128), lambda g: (g, 0)),
    out_shape=jax.ShapeDtypeStruct((_SPLIT, 128), jnp.float32),
)

_mesh = plsc.VectorSubcoreMesh(core_axis_name="c", subcore_axis_name="s")


@functools.partial(
    pl.kernel,
    out_type=jax.ShapeDtypeStruct((_J, _BT, 128, 128), jnp.float32),
    mesh=_mesh,
    scratch_types=(
        [pltpu.VMEM((_J, 128), jnp.int32)]
        + [pltpu.VMEM((128,), jnp.int32)] * _NB
        + [pltpu.VMEM((128, 128), jnp.float32)] * _NB
        + [pltpu.SemaphoreType.DMA] * (2 * _NB)
    ),
)
def _gather_rows(idx_hbm, table_hbm, out_hbm, idx_v, *rest):
    idx_g = rest[:_NB]
    buf_g = rest[_NB:2 * _NB]
    sem_g = rest[2 * _NB:3 * _NB]
    sem_s = rest[3 * _NB:4 * _NB]

    wid = lax.axis_index("s") * _NC + lax.axis_index("c")

    pltpu.sync_copy(idx_hbm.at[:, wid], idx_v)

    def stage_and_issue_gather(j, b):
        for g in range(8):
            sl = pl.ds(16 * g, 16)
            idx_g[b][sl] = idx_v[j, sl] & (_SPLIT - 1)
        pltpu.async_copy(table_hbm.at[idx_g[b]], buf_g[b], sem_g[b])

    def wait_gather(b):
        pltpu.make_async_copy(
            table_hbm.at[pl.ds(0, 128)], buf_g[b], sem_g[b]
        ).wait()

    def wait_scatter(b):
        pltpu.make_async_copy(buf_g[b], out_hbm.at[0, 0], sem_s[b]).wait()

    for b in range(_NB):
        stage_and_issue_gather(b, b)

    @pl.loop(0, _JG)
    def _group(t):
        for k in range(_NB):
            j = t * _NB + k
            wait_gather(k)
            pltpu.async_copy(buf_g[k], out_hbm.at[j, wid], sem_s[k])

            @pl.when(t + 1 < _JG)
            def _():
                wait_scatter(k)
                stage_and_issue_gather(j + _NB, k)

    for k in range(_NB):
        wait_scatter(k)


_JB = 4


def _transpose_body(i_ref, idx_ref, o_ref):
    for jj in range(_JB):
        sels = []
        for bt in range(_BT):
            t = i_ref[jj, bt, :, :].T
            m = (idx_ref[jj, bt, :] >= _SPLIT)[None, :]
            sel = jnp.where(m, t[64:, :], t[:64, :]) * _SCALE
            sels.append(sel.reshape(8, 1, 8, 128))
        o_ref[jj] = jnp.concatenate(sels, axis=1)


_to_final = pl.pallas_call(
    _transpose_body,
    grid=(_J // _JB,),
    in_specs=[
        pl.BlockSpec((_JB, _BT, 128, 128), lambda j: (j, 0, 0, 0)),
        pl.BlockSpec((_JB, _BT, 128), lambda j: (j, 0, 0)),
    ],
    out_specs=pl.BlockSpec(
        (_JB, 8, _BT, 8, 128), lambda j: (j, 0, 0, 0, 0)
    ),
    out_shape=jax.ShapeDtypeStruct((_J, 8, _BT, 8, 128), jnp.float32),
)


def kernel(x, table):
    table2 = _pack_table(table.T, table.T)
    idx = x.T.reshape(_J, _BT, 128)
    out_sc = _gather_rows(idx, table2)
    out5 = _to_final(out_sc, idx)
    return out5.transpose(2, 4, 0, 1, 3).reshape(4096, _J, _HIDDEN)


# speedup vs baseline: 1.8235x; 1.0012x over previous
import functools

import jax
import jax.numpy as jnp
from jax import lax
from jax.experimental import pallas as pl
from jax.experimental.pallas import tpu as pltpu
from jax.experimental.pallas import tpu_sc as plsc

_HIDDEN = 64
_SCALE = 8.0
_VOCAB = 1000000
_SPLIT = 524288
_NC = 2
_NS = 16
_NW = _NC * _NS
_BT = 4096 // 128
_J = 200
_NB = 5
_JG = _J // _NB

_P1W = 8192
_P1G = _SPLIT // _P1W
_P1LAST = (_VOCAB - 1) // _P1W


def _pack_body(a_ref, b_ref, o_ref):
    o_ref[...] = jnp.concatenate([a_ref[...], b_ref[...]], axis=0).T


_pack_table = pl.pallas_call(
    _pack_body,
    grid=(_P1G,),
    in_specs=[
        pl.BlockSpec((_HIDDEN, _P1W), lambda g: (0, g)),
        pl.BlockSpec(
            (_HIDDEN, _P1W),
            lambda g: (0, jnp.minimum(_P1G + g, _P1LAST)),
        ),
    ],
    out_specs=pl.BlockSpec((_P1W, 128), lambda g: (g, 0)),
    out_shape=jax.ShapeDtypeStruct((_SPLIT, 128), jnp.float32),
)

_mesh = plsc.VectorSubcoreMesh(core_axis_name="c", subcore_axis_name="s")


@functools.partial(
    pl.kernel,
    out_type=jax.ShapeDtypeStruct((_J, _BT, 128, 128), jnp.float32),
    mesh=_mesh,
    scratch_types=(
        [pltpu.VMEM((_J, 128), jnp.int32)]
        + [pltpu.VMEM((128,), jnp.int32)] * _NB
        + [pltpu.VMEM((128, 128), jnp.float32)] * _NB
        + [pltpu.SemaphoreType.DMA] * (2 * _NB)
    ),
)
def _gather_rows(idx_hbm, table_hbm, out_hbm, idx_v, *rest):
    idx_g = rest[:_NB]
    buf_g = rest[_NB:2 * _NB]
    sem_g = rest[2 * _NB:3 * _NB]
    sem_s = rest[3 * _NB:4 * _NB]

    wid = lax.axis_index("s") * _NC + lax.axis_index("c")

    pltpu.sync_copy(idx_hbm.at[:, wid], idx_v)

    def stage_and_issue_gather(j, b):
        for g in range(8):
            sl = pl.ds(16 * g, 16)
            idx_g[b][sl] = idx_v[j, sl] & (_SPLIT - 1)
        pltpu.async_copy(table_hbm.at[idx_g[b]], buf_g[b], sem_g[b])

    def wait_gather(b):
        pltpu.make_async_copy(
            table_hbm.at[pl.ds(0, 128)], buf_g[b], sem_g[b]
        ).wait()

    def wait_scatter(b):
        pltpu.make_async_copy(buf_g[b], out_hbm.at[0, 0], sem_s[b]).wait()

    for b in range(_NB):
        stage_and_issue_gather(b, b)

    @pl.loop(0, _JG)
    def _group(t):
        for k in range(_NB):
            j = t * _NB + k
            wait_gather(k)
            pltpu.async_copy(buf_g[k], out_hbm.at[j, wid], sem_s[k])

            @pl.when(t + 1 < _JG)
            def _():
                wait_scatter(k)
                stage_and_issue_gather(j + _NB, k)

    for k in range(_NB):
        wait_scatter(k)


_JB = 8


def _transpose_body(i_ref, idx_ref, o_ref):
    for jj in range(_JB):
        sels = []
        for bt in range(_BT):
            t = i_ref[jj, bt, :, :].T
            m = (idx_ref[jj, bt, :] >= _SPLIT)[None, :]
            sel = jnp.where(m, t[64:, :], t[:64, :]) * _SCALE
            sels.append(sel.reshape(8, 1, 8, 128))
        o_ref[jj] = jnp.concatenate(sels, axis=1)


_to_final = pl.pallas_call(
    _transpose_body,
    grid=(_J // _JB,),
    in_specs=[
        pl.BlockSpec((_JB, _BT, 128, 128), lambda j: (j, 0, 0, 0)),
        pl.BlockSpec((_JB, _BT, 128), lambda j: (j, 0, 0)),
    ],
    out_specs=pl.BlockSpec(
        (_JB, 8, _BT, 8, 128), lambda j: (j, 0, 0, 0, 0)
    ),
    out_shape=jax.ShapeDtypeStruct((_J, 8, _BT, 8, 128), jnp.float32),
)


def kernel(x, table):
    table2 = _pack_table(table.T, table.T)
    idx = x.T.reshape(_J, _BT, 128)
    out_sc = _gather_rows(idx, table2)
    out5 = _to_final(out_sc, idx)
    return out5.transpose(2, 4, 0, 1, 3).reshape(4096, _J, _HIDDEN)
